# Initial kernel scaffold; baseline (speedup 1.0000x reference)
import jax
import jax.numpy as jnp
from jax.experimental import pallas as pl


def kernel(mlmm_distances, mlmm_vectors, mlmm_atomic_charges, atomic_dipoles, mlmm_idxu, mlmm_idxv):
    raise NotImplementedError("write your pallas kernel here")


# speedup vs baseline: 4.6544x; 4.6544x over previous
import functools

import jax
import jax.numpy as jnp
from jax import lax
from jax.experimental import pallas as pl
from jax.experimental.pallas import tpu as pltpu
from jax.experimental.pallas import tpu_sc as plsc

CUTOFF = 12.0
CUTON = 0.8 * CUTOFF
KE = 14.399645

N_NODES = 50000
N_EDGES = 1600000
NW = 32
E_PER_W = N_EDGES // NW
B = 2000
NCH = E_PER_W // B
LANES = 16


def _body(d_hbm, vec_hbm, q_hbm, dx_hbm, dy_hbm, dz_hbm, iu_hbm, iv_hbm,
          out_hbm,
          iu_v, iv_v, d_v, vec_v, qu_v, qv_v, dx_v, dy_v, dz_v, o_v, sem):
    wid = lax.axis_index("s") * 2 + lax.axis_index("c")

    c_shift_a = 2.0 / CUTOFF
    c_shift_b = 1.0 / (CUTOFF * CUTOFF)
    inv_w = 1.0 / (CUTOFF - CUTON)

    def chunk_body(ci, carry):
        base = wid * E_PER_W + ci * B
        pltpu.sync_copy(iu_hbm.at[pl.ds(base, B)], iu_v)
        pltpu.sync_copy(iv_hbm.at[pl.ds(base, B)], iv_v)
        pltpu.sync_copy(d_hbm.at[pl.ds(base, B)], d_v)
        pltpu.sync_copy(vec_hbm.at[pl.ds(3 * base, 3 * B)], vec_v)
        pltpu.async_copy(q_hbm.at[iu_v], qu_v, sem).wait()
        pltpu.async_copy(q_hbm.at[iv_v], qv_v, sem).wait()
        pltpu.async_copy(dx_hbm.at[iu_v], dx_v, sem).wait()
        pltpu.async_copy(dy_hbm.at[iu_v], dy_v, sem).wait()
        pltpu.async_copy(dz_hbm.at[iu_v], dz_v, sem).wait()

        def step(i, carry2):
            s = i * LANES
            rows3 = 3 * s + 3 * lax.iota(jnp.int32, LANES)
            d = d_v[pl.ds(s, LANES)]
            qu = qu_v[pl.ds(s, LANES)]
            qv = qv_v[pl.ds(s, LANES)]
            dx = dx_v[pl.ds(s, LANES)]
            dy = dy_v[pl.ds(s, LANES)]
            dz = dz_v[pl.ds(s, LANES)]
            vx = plsc.load_gather(vec_v, [rows3])
            vy = plsc.load_gather(vec_v, [rows3 + 1])
            vz = plsc.load_gather(vec_v, [rows3 + 2])

            chi = 1.0 / d
            chi_shift = c_shift_a - d * c_shift_b
            e = qu * qv * (chi - chi_shift)
            chi2 = chi * chi
            chi2_shift = chi_shift * chi_shift
            dot = (vx * dx + vy * dy + vz * dz) * chi
            e = e + qv * dot * (chi2 - chi2_shift)
            x = (d - CUTON) * inv_w
            x = jnp.minimum(jnp.maximum(x, 0.0), 1.0)
            sw = 1.0 + x * x * x * (-10.0 + x * (15.0 - 6.0 * x))
            o_v[pl.ds(s, LANES)] = (KE * e) * sw
            return carry2

        lax.fori_loop(0, B // LANES, step, 0)
        pltpu.sync_copy(o_v, out_hbm.at[pl.ds(base, B)])
        return carry

    lax.fori_loop(0, NCH, chunk_body, 0)


def kernel(mlmm_distances, mlmm_vectors, mlmm_atomic_charges, atomic_dipoles,
           mlmm_idxu, mlmm_idxv):
    mesh = plsc.VectorSubcoreMesh(core_axis_name="c", subcore_axis_name="s")
    run = functools.partial(
        pl.kernel,
        out_type=jax.ShapeDtypeStruct((N_EDGES,), jnp.float32),
        mesh=mesh,
        compiler_params=pltpu.CompilerParams(needs_layout_passes=False),
        scratch_types=[
            pltpu.VMEM((B,), jnp.int32),
            pltpu.VMEM((B,), jnp.int32),
            pltpu.VMEM((B,), jnp.float32),
            pltpu.VMEM((3 * B,), jnp.float32),
            pltpu.VMEM((B,), jnp.float32),
            pltpu.VMEM((B,), jnp.float32),
            pltpu.VMEM((B,), jnp.float32),
            pltpu.VMEM((B,), jnp.float32),
            pltpu.VMEM((B,), jnp.float32),
            pltpu.VMEM((B,), jnp.float32),
            pltpu.SemaphoreType.DMA,
        ],
    )(_body)
    return run(mlmm_distances, mlmm_vectors.reshape(-1), mlmm_atomic_charges,
               atomic_dipoles[:, 0], atomic_dipoles[:, 1],
               atomic_dipoles[:, 2], mlmm_idxu.astype(jnp.int32),
               mlmm_idxv.astype(jnp.int32))
